# baseline (speedup 1.0000x reference)
import functools

import jax
import jax.numpy as jnp
from jax import lax
from jax.experimental import pallas as pl
from jax.experimental.pallas import tpu as pltpu
from jax.experimental.pallas import tpu_sc as plsc

NC = 2
NS = 16
NW = NC * NS
CH = 128

_f32 = jnp.float32


def _sc_gather_pos(pos16, src, dst):
  e = src.shape[0]
  ew = e // NW
  nfull = ew // CH
  tail = ew - nfull * CH
  ngrp = nfull // NB
  assert ngrp * NB == nfull and ngrp % 2 == 1
  mesh = plsc.VectorSubcoreMesh(core_axis_name="c", subcore_axis_name="s")
  nbuf = NSET * NB

  @functools.partial(
      pl.kernel,
      out_type=[
          jax.ShapeDtypeStruct((e, 16), _f32),
          jax.ShapeDtypeStruct((e, 16), _f32),
      ],
      mesh=mesh,
      scratch_types=(
          [pltpu.VMEM((ew,), jnp.int32), pltpu.VMEM((ew,), jnp.int32)]
          + [pltpu.VMEM((CH, 16), _f32) for _ in range(2 * nbuf)]
          + [pltpu.VMEM((max(tail, 8), 16), _f32),
             pltpu.VMEM((max(tail, 8), 16), _f32),
             pltpu.SemaphoreType.DMA,
             pltpu.SemaphoreType.DMA,
             pltpu.SemaphoreType.DMA]
      ),
      compiler_params=pltpu.CompilerParams(use_tc_tiling_on_sc=False),
  )
  def gk(pos_hbm, src_hbm, dst_hbm, pj_hbm, pi_hbm, idx_s, idx_d, *rest):
    jbufs = rest[:nbuf]
    ibufs = rest[nbuf:2 * nbuf]
    rtj, rti, gsem, wsem, tsem = rest[2 * nbuf:]
    wid = lax.axis_index("s") * NC + lax.axis_index("c")
    base = wid * ew
    pltpu.sync_copy(src_hbm.at[pl.ds(base, ew)], idx_s)
    pltpu.sync_copy(dst_hbm.at[pl.ds(base, ew)], idx_d)
    jsets = (jbufs[:NB], jbufs[NB:])
    isets = (ibufs[:NB], ibufs[NB:])

    def fire_gathers(g, p):
      for b in range(NB):
        c = g * NB + b
        pltpu.async_copy(pos_hbm.at[idx_s.at[pl.ds(c * CH, CH)]],
                         jsets[p][b], gsem)
        pltpu.async_copy(pos_hbm.at[idx_d.at[pl.ds(c * CH, CH)]],
                         isets[p][b], gsem)

    def wait_gathers(p):
      for b in range(NB):
        pltpu.make_async_copy(pj_hbm.at[pl.ds(0, CH)], jsets[p][b],
                              gsem).wait()
        pltpu.make_async_copy(pj_hbm.at[pl.ds(0, CH)], isets[p][b],
                              gsem).wait()

    def fire_writes(g, p):
      for b in range(NB):
        c = g * NB + b
        pltpu.async_copy(jsets[p][b],
                         pj_hbm.at[pl.ds(base + c * CH, CH)], wsem)
        pltpu.async_copy(isets[p][b],
                         pi_hbm.at[pl.ds(base + c * CH, CH)], wsem)

    def drain_writes(p):
      for b in range(NB):
        pltpu.make_async_copy(jsets[p][b], pj_hbm.at[pl.ds(0, CH)],
                              wsem).wait()
        pltpu.make_async_copy(isets[p][b], pj_hbm.at[pl.ds(0, CH)],
                              wsem).wait()

    fire_gathers(0, 0)
    wait_gathers(0)
    fire_gathers(1, 1)
    fire_writes(0, 0)

    def body(k, carry):
      g1 = 2 * k + 1
      wait_gathers(1)
      drain_writes(0)
      fire_gathers(g1 + 1, 0)
      fire_writes(g1, 1)
      g2 = 2 * k + 2
      wait_gathers(0)

      @pl.when(g2 + 1 < ngrp)
      def _():
        drain_writes(1)
        fire_gathers(g2 + 1, 1)

      fire_writes(g2, 0)
      return carry

    lax.fori_loop(0, (ngrp - 1) // 2, body, 0)

    if tail:
      ta = pltpu.async_copy(
          pos_hbm.at[idx_s.at[pl.ds(nfull * CH, tail)]],
          rtj.at[pl.ds(0, tail)], tsem)
      tb = pltpu.async_copy(
          pos_hbm.at[idx_d.at[pl.ds(nfull * CH, tail)]],
          rti.at[pl.ds(0, tail)], tsem)
      ta.wait()
      tb.wait()
      pltpu.sync_copy(rtj.at[pl.ds(0, tail)],
                      pj_hbm.at[pl.ds(base + nfull * CH, tail)])
      pltpu.sync_copy(rti.at[pl.ds(0, tail)],
                      pi_hbm.at[pl.ds(base + nfull * CH, tail)])
    for p in range(NSET):
      drain_writes(p)

  return gk(pos16, src, dst)


NB = 3
NSET = 2


def _sc_gather1(x, src):
  n, d = x.shape
  e = src.shape[0]
  ew = e // NW
  nfull = ew // CH
  tail = ew - nfull * CH
  ngrp = nfull // NB
  assert ngrp * NB == nfull and ngrp % 2 == 1
  mesh = plsc.VectorSubcoreMesh(core_axis_name="c", subcore_axis_name="s")
  nbuf = NSET * NB

  @functools.partial(
      pl.kernel,
      out_type=jax.ShapeDtypeStruct((e, d), _f32),
      mesh=mesh,
      scratch_types=(
          [pltpu.VMEM((ew,), jnp.int32)]
          + [pltpu.VMEM((CH, d), _f32) for _ in range(nbuf)]
          + [pltpu.VMEM((max(tail, 8), d), _f32),
             pltpu.SemaphoreType.DMA,
             pltpu.SemaphoreType.DMA,
             pltpu.SemaphoreType.DMA]
      ),
  )
  def gk(x_hbm, src_hbm, xj_hbm, idx_all, *rest):
    bufs = rest[:nbuf]
    rt, gsem, wsem, tsem = rest[nbuf:]
    wid = lax.axis_index("s") * NC + lax.axis_index("c")
    base = wid * ew
    pltpu.sync_copy(src_hbm.at[pl.ds(base, ew)], idx_all)
    sets = (bufs[:NB], bufs[NB:])

    def fire_gathers(g, sbufs):
      for b in range(NB):
        c = g * NB + b
        pltpu.async_copy(x_hbm.at[idx_all.at[pl.ds(c * CH, CH)]],
                         sbufs[b], gsem)

    def wait_gathers(sbufs):
      for b in range(NB):
        pltpu.make_async_copy(xj_hbm.at[pl.ds(0, CH)], sbufs[b], gsem).wait()

    def fire_writes(g, sbufs):
      for b in range(NB):
        c = g * NB + b
        pltpu.async_copy(sbufs[b], xj_hbm.at[pl.ds(base + c * CH, CH)], wsem)

    def drain_writes(sbufs):
      for b in range(NB):
        pltpu.make_async_copy(sbufs[b], xj_hbm.at[pl.ds(0, CH)], wsem).wait()

    fire_gathers(0, sets[0])
    wait_gathers(sets[0])
    fire_gathers(1, sets[1])
    fire_writes(0, sets[0])

    def body(k, carry):
      g1 = 2 * k + 1
      wait_gathers(sets[1])
      drain_writes(sets[0])
      fire_gathers(g1 + 1, sets[0])
      fire_writes(g1, sets[1])
      g2 = 2 * k + 2
      wait_gathers(sets[0])

      @pl.when(g2 + 1 < ngrp)
      def _():
        drain_writes(sets[1])
        fire_gathers(g2 + 1, sets[1])

      fire_writes(g2, sets[0])
      return carry

    lax.fori_loop(0, (ngrp - 1) // 2, body, 0)

    if tail:
      pltpu.async_copy(
          x_hbm.at[idx_all.at[pl.ds(nfull * CH, tail)]],
          rt.at[pl.ds(0, tail)], tsem).wait()
      pltpu.sync_copy(rt.at[pl.ds(0, tail)],
                      xj_hbm.at[pl.ds(base + nfull * CH, tail)])
    for sbufs in sets:
      drain_writes(sbufs)

  return gk(x, src)


def _sc_scatter_both(ma, mb, dst, n):
  e, da = ma.shape
  db = mb.shape[1]
  dha = da // NC
  dhb = db // NC
  ept = e // NS
  nfull = ept // CH
  etail = ept - nfull * CH
  nb = 2
  ngrp = nfull // nb
  assert ngrp * nb == nfull and ngrp % 2 == 1
  br = (n // NS) // 8 * 8
  ntail = n - br * NS
  mesh = plsc.VectorSubcoreMesh(core_axis_name="c", subcore_axis_name="s")
  nbuf = NSET * nb

  zeros_a = jnp.zeros((max(br, ntail), dha), _f32)
  zeros_b = jnp.zeros((max(br, ntail), dhb), _f32)

  @functools.partial(
      pl.kernel,
      out_type=[
          jax.ShapeDtypeStruct((NC, n, dha), _f32),
          jax.ShapeDtypeStruct((NC, n, dhb), _f32),
      ],
      mesh=mesh,
      scratch_types=(
          [pltpu.VMEM((CH,), jnp.int32) for _ in range(nbuf)]
          + [pltpu.VMEM((CH, dha), _f32) for _ in range(nbuf)]
          + [pltpu.VMEM((CH, dhb), _f32) for _ in range(nbuf)]
          + [pltpu.VMEM((max(etail, 8),), jnp.int32),
             pltpu.VMEM((max(etail, 8), dha), _f32),
             pltpu.VMEM((max(etail, 8), dhb), _f32),
             pltpu.VMEM_SHARED((n, dha), _f32),
             pltpu.VMEM_SHARED((n, dhb), _f32),
             pltpu.SemaphoreType.DMA,
             pltpu.SemaphoreType.DMA,
             pltpu.SemaphoreType.DMA]
      ),
      compiler_params=pltpu.CompilerParams(use_tc_tiling_on_sc=False),
  )
  def sk(ma_hbm, mb_hbm, dst_hbm, za_hbm, zb_hbm, pa_hbm, pb_hbm, *rest):
    ibufs = rest[:nbuf]
    abufs = rest[nbuf:2 * nbuf]
    bbufs = rest[2 * nbuf:3 * nbuf]
    ti, ta_, tb_, acc_a, acc_b, lsem, ssem, tsem = rest[3 * nbuf:]
    cid = lax.axis_index("c")
    sid = lax.axis_index("s")
    base = sid * ept
    r0 = sid * br
    pltpu.sync_copy(za_hbm.at[pl.ds(0, br)], acc_a.at[pl.ds(r0, br)])
    pltpu.sync_copy(zb_hbm.at[pl.ds(0, br)], acc_b.at[pl.ds(r0, br)])

    @pl.when(sid == NS - 1)
    def _():
      pltpu.sync_copy(za_hbm.at[pl.ds(0, ntail)],
                      acc_a.at[pl.ds(NS * br, ntail)])
      pltpu.sync_copy(zb_hbm.at[pl.ds(0, ntail)],
                      acc_b.at[pl.ds(NS * br, ntail)])

    plsc.subcore_barrier()

    isets = (ibufs[:nb], ibufs[nb:])
    asets = (abufs[:nb], abufs[nb:])
    bsets = (bbufs[:nb], bbufs[nb:])

    def fire_loads(g, p):
      for b in range(nb):
        c = g * nb + b
        pltpu.async_copy(dst_hbm.at[pl.ds(base + c * CH, CH)],
                         isets[p][b], lsem)
        pltpu.async_copy(
            ma_hbm.at[pl.ds(base + c * CH, CH), pl.ds(cid * dha, dha)],
            asets[p][b], lsem)
        pltpu.async_copy(
            mb_hbm.at[pl.ds(base + c * CH, CH), pl.ds(cid * dhb, dhb)],
            bsets[p][b], lsem)

    def wait_loads(p):
      for b in range(nb):
        pltpu.make_async_copy(dst_hbm.at[pl.ds(0, CH)], isets[p][b],
                              lsem).wait()
        pltpu.make_async_copy(ma_hbm.at[pl.ds(0, CH), pl.ds(0, dha)],
                              asets[p][b], lsem).wait()
        pltpu.make_async_copy(mb_hbm.at[pl.ds(0, CH), pl.ds(0, dhb)],
                              bsets[p][b], lsem).wait()

    def fire_adds(p):
      for b in range(nb):
        pltpu.async_copy(asets[p][b], acc_a.at[isets[p][b]], ssem, add=True)
        pltpu.async_copy(bsets[p][b], acc_b.at[isets[p][b]], ssem, add=True)

    def drain_adds(p):
      for b in range(nb):
        pltpu.make_async_copy(asets[p][b], acc_a.at[pl.ds(0, CH)],
                              ssem).wait()
        pltpu.make_async_copy(bsets[p][b], acc_b.at[pl.ds(0, CH)],
                              ssem).wait()

    fire_loads(0, 0)
    wait_loads(0)
    fire_loads(1, 1)
    fire_adds(0)

    def body(k, carry):
      wait_loads(1)
      drain_adds(0)
      fire_loads(2 * k + 2, 0)
      fire_adds(1)
      wait_loads(0)

      @pl.when(2 * k + 3 < ngrp)
      def _():
        drain_adds(1)
        fire_loads(2 * k + 3, 1)

      fire_adds(0)
      return carry

    lax.fori_loop(0, (ngrp - 1) // 2, body, 0)

    if etail:
      t1 = pltpu.async_copy(dst_hbm.at[pl.ds(base + nfull * CH, etail)],
                            ti.at[pl.ds(0, etail)], tsem)
      t2 = pltpu.async_copy(
          ma_hbm.at[pl.ds(base + nfull * CH, etail), pl.ds(cid * dha, dha)],
          ta_.at[pl.ds(0, etail)], tsem)
      t3 = pltpu.async_copy(
          mb_hbm.at[pl.ds(base + nfull * CH, etail), pl.ds(cid * dhb, dhb)],
          tb_.at[pl.ds(0, etail)], tsem)
      t1.wait()
      t2.wait()
      t3.wait()
      pltpu.sync_copy(ta_.at[pl.ds(0, etail)], acc_a.at[ti], add=True)
      pltpu.sync_copy(tb_.at[pl.ds(0, etail)], acc_b.at[ti], add=True)
    for p in range(NSET):
      drain_adds(p)
    plsc.subcore_barrier()
    pltpu.sync_copy(acc_a.at[pl.ds(r0, br)], pa_hbm.at[cid, pl.ds(r0, br)])
    pltpu.sync_copy(acc_b.at[pl.ds(r0, br)], pb_hbm.at[cid, pl.ds(r0, br)])

    @pl.when(sid == NS - 1)
    def _():
      pltpu.sync_copy(acc_a.at[pl.ds(NS * br, ntail)],
                      pa_hbm.at[cid, pl.ds(NS * br, ntail)])
      pltpu.sync_copy(acc_b.at[pl.ds(NS * br, ntail)],
                      pb_hbm.at[cid, pl.ds(NS * br, ntail)])

  return sk(ma, mb, dst, zeros_a, zeros_b)


def _edge_body(xj_ref, pi_ref, pj_ref, wpi_ref, wpj_ref, w9_ref, ppb_ref,
               awa_ref, awb_ref, ab_ref, oa_ref, ob_ref):
  xj = xj_ref[...]
  pi = pi_ref[...]
  pj = pj_ref[...]
  v = pi - pj
  dij = jnp.sqrt(jnp.sum(v * v, axis=1, keepdims=True))
  r = pi @ wpi_ref[...] + pj @ wpj_ref[...] + dij * w9_ref[...] + ppb_ref[...]
  r = jnp.maximum(r, 0.0)
  g = xj @ awa_ref[...] + r @ awb_ref[...] + ab_ref[...]
  g = jnp.maximum(g, 0.0)
  m = jnp.max(g, axis=1, keepdims=True)
  eg = jnp.exp(g - m)
  s = eg / jnp.sum(eg, axis=1, keepdims=True)
  da = xj.shape[1]
  oa_ref[...] = s[:, :da] * xj
  ob_ref[...] = s[:, da:] * r


def _tc_edge(xj, pi16, pj16, wpi, wpj, w9, ppb, awa, awb, ab, block=1000):
  e, d = xj.shape
  dp = wpi.shape[1]
  df = d + dp
  dh = d // 2
  grid = e // block
  full = lambda i: (0, 0)
  return pl.pallas_call(
      _edge_body,
      grid=(grid,),
      in_specs=[
          pl.BlockSpec((block, d), lambda i: (i, 0)),
          pl.BlockSpec((block, 16), lambda i: (i, 0)),
          pl.BlockSpec((block, 16), lambda i: (i, 0)),
          pl.BlockSpec((16, dp), full),
          pl.BlockSpec((16, dp), full),
          pl.BlockSpec((1, dp), full),
          pl.BlockSpec((1, dp), full),
          pl.BlockSpec((d, df), full),
          pl.BlockSpec((dp, df), full),
          pl.BlockSpec((1, df), full),
      ],
      out_specs=[
          pl.BlockSpec((block, d), lambda i: (i, 0)),
          pl.BlockSpec((block, dp), lambda i: (i, 0)),
      ],
      out_shape=[
          jax.ShapeDtypeStruct((e, d), _f32),
          jax.ShapeDtypeStruct((e, dp), _f32),
      ],
  )(xj, pi16, pj16, wpi, wpj, w9, ppb, awa, awb, ab)


def _global_body(pa_ref, pb_ref, gwa0_ref, gwa1_ref, gwb0_ref, gwb1_ref,
                 gb_ref, o_ref):
  o_ref[...] = jnp.maximum(
      pa_ref[0] @ gwa0_ref[...] + pa_ref[1] @ gwa1_ref[...]
      + pb_ref[0] @ gwb0_ref[...] + pb_ref[1] @ gwb1_ref[...]
      + gb_ref[...], 0.0)


def _tc_global(pa, pb, gwa0, gwa1, gwb0, gwb1, gb, block=1000):
  _, n, dh = pa.shape
  db = pb.shape[2]
  dout = gwa0.shape[1]
  grid = n // block
  full = lambda i: (0, 0)
  return pl.pallas_call(
      _global_body,
      grid=(grid,),
      in_specs=[
          pl.BlockSpec((NC, block, dh), lambda i: (0, i, 0)),
          pl.BlockSpec((NC, block, db), lambda i: (0, i, 0)),
          pl.BlockSpec((dh, dout), full),
          pl.BlockSpec((dh, dout), full),
          pl.BlockSpec((db, dout), full),
          pl.BlockSpec((db, dout), full),
          pl.BlockSpec((1, dout), full),
      ],
      out_specs=pl.BlockSpec((block, dout), lambda i: (i, 0)),
      out_shape=jax.ShapeDtypeStruct((n, dout), _f32),
  )(pa, pb, gwa0, gwa1, gwb0, gwb1, gb)


def _global_res_body(pa_ref, pb_ref, x_ref, gwa0_ref, gwa1_ref, gwb0_ref,
                     gwb1_ref, gb_ref, scw_ref, scb_ref, o_ref):
  h = (pa_ref[0] @ gwa0_ref[...] + pa_ref[1] @ gwa1_ref[...]
       + pb_ref[0] @ gwb0_ref[...] + pb_ref[1] @ gwb1_ref[...]
       + gb_ref[...])
  h = jnp.maximum(h, 0.0)
  sc = x_ref[...] @ scw_ref[...] + scb_ref[...]
  o_ref[...] = jnp.maximum(h + sc, 0.0)


def _tc_global_res(pa, pb, x, gwa0, gwa1, gwb0, gwb1, gb, scw, scb,
                   block=1000):
  _, n, dh = pa.shape
  db = pb.shape[2]
  d = x.shape[1]
  dout = gwa0.shape[1]
  grid = n // block
  full = lambda i: (0, 0)
  return pl.pallas_call(
      _global_res_body,
      grid=(grid,),
      in_specs=[
          pl.BlockSpec((NC, block, dh), lambda i: (0, i, 0)),
          pl.BlockSpec((NC, block, db), lambda i: (0, i, 0)),
          pl.BlockSpec((block, d), lambda i: (i, 0)),
          pl.BlockSpec((dh, dout), full),
          pl.BlockSpec((dh, dout), full),
          pl.BlockSpec((db, dout), full),
          pl.BlockSpec((db, dout), full),
          pl.BlockSpec((1, dout), full),
          pl.BlockSpec((d, dout), full),
          pl.BlockSpec((1, dout), full),
      ],
      out_specs=pl.BlockSpec((block, dout), lambda i: (i, 0)),
      out_shape=jax.ShapeDtypeStruct((n, dout), _f32),
  )(pa, pb, x, gwa0, gwa1, gwb0, gwb1, gb, scw, scb)


def _prep_pp(ppW):
  wpi = jnp.zeros((16, ppW.shape[1]), _f32).at[:3].set(ppW[0:3] + ppW[6:9])
  wpj = jnp.zeros((16, ppW.shape[1]), _f32).at[:3].set(ppW[3:6] - ppW[6:9])
  w9 = ppW[9:10]
  return wpi, wpj, w9


def kernel(x, pos, edge_index, ppW1, ppb1, aW1, ab1, gW1, gb1,
           ppW2, ppb2, aW2, ab2, gW2, gb2, scW, scb):
  n, d = x.shape
  src = edge_index[0]
  dst = edge_index[1]
  pos16 = jnp.zeros((n, 16), _f32).at[:, :3].set(pos)

  pj16, pi16 = _sc_gather_pos(pos16, src, dst)
  xj = _sc_gather1(x, src)

  dh = d // 2
  dq = d + (192 - d) // 2
  wpi1, wpj1, w91 = _prep_pp(ppW1)
  ma1, mb1 = _tc_edge(xj, pi16, pj16, wpi1, wpj1, w91, ppb1[None, :],
                      aW1[:d], aW1[d:], ab1[None, :])
  pa1, pb1 = _sc_scatter_both(ma1, mb1, dst, n)
  h1 = _tc_global(pa1, pb1, gW1[:dh], gW1[dh:d], gW1[d:dq], gW1[dq:],
                  gb1[None, :])

  hj = _sc_gather1(h1, src)
  wpi2, wpj2, w92 = _prep_pp(ppW2)
  ma2, mb2 = _tc_edge(hj, pi16, pj16, wpi2, wpj2, w92, ppb2[None, :],
                      aW2[:d], aW2[d:], ab2[None, :])
  pa2, pb2 = _sc_scatter_both(ma2, mb2, dst, n)
  out = _tc_global_res(pa2, pb2, x, gW2[:dh], gW2[dh:d], gW2[d:dq],
                       gW2[dq:], gb2[None, :], scW, scb[None, :])
  return out


# speedup vs baseline: 1.0656x; 1.0656x over previous
import functools

import jax
import jax.numpy as jnp
from jax import lax
from jax.experimental import pallas as pl
from jax.experimental.pallas import tpu as pltpu
from jax.experimental.pallas import tpu_sc as plsc

NC = 2
NS = 16
NW = NC * NS
CH = 128

_f32 = jnp.float32


def _sc_gather_pos(pos16, src, dst):
  e = src.shape[0]
  ew = e // NW
  nfull = ew // CH
  tail = ew - nfull * CH
  ngrp = nfull // NB
  assert ngrp * NB == nfull and ngrp % 2 == 1
  mesh = plsc.VectorSubcoreMesh(core_axis_name="c", subcore_axis_name="s")
  nbuf = NSET * NB

  @functools.partial(
      pl.kernel,
      out_type=[
          jax.ShapeDtypeStruct((e, 16), _f32),
          jax.ShapeDtypeStruct((e, 16), _f32),
      ],
      mesh=mesh,
      scratch_types=(
          [pltpu.VMEM((ew,), jnp.int32), pltpu.VMEM((ew,), jnp.int32)]
          + [pltpu.VMEM((CH, 16), _f32) for _ in range(2 * nbuf)]
          + [pltpu.VMEM((max(tail, 8), 16), _f32),
             pltpu.VMEM((max(tail, 8), 16), _f32),
             pltpu.SemaphoreType.DMA,
             pltpu.SemaphoreType.DMA,
             pltpu.SemaphoreType.DMA]
      ),
      compiler_params=pltpu.CompilerParams(use_tc_tiling_on_sc=False),
  )
  def gk(pos_hbm, src_hbm, dst_hbm, pj_hbm, pi_hbm, idx_s, idx_d, *rest):
    jbufs = rest[:nbuf]
    ibufs = rest[nbuf:2 * nbuf]
    rtj, rti, gsem, wsem, tsem = rest[2 * nbuf:]
    wid = lax.axis_index("s") * NC + lax.axis_index("c")
    base = wid * ew
    pltpu.sync_copy(src_hbm.at[pl.ds(base, ew)], idx_s)
    pltpu.sync_copy(dst_hbm.at[pl.ds(base, ew)], idx_d)
    jsets = (jbufs[:NB], jbufs[NB:])
    isets = (ibufs[:NB], ibufs[NB:])

    def fire_gathers(g, p):
      for b in range(NB):
        c = g * NB + b
        pltpu.async_copy(pos_hbm.at[idx_s.at[pl.ds(c * CH, CH)]],
                         jsets[p][b], gsem)
        pltpu.async_copy(pos_hbm.at[idx_d.at[pl.ds(c * CH, CH)]],
                         isets[p][b], gsem)

    def wait_gathers(p):
      for b in range(NB):
        pltpu.make_async_copy(pj_hbm.at[pl.ds(0, CH)], jsets[p][b],
                              gsem).wait()
        pltpu.make_async_copy(pj_hbm.at[pl.ds(0, CH)], isets[p][b],
                              gsem).wait()

    def fire_writes(g, p):
      for b in range(NB):
        c = g * NB + b
        pltpu.async_copy(jsets[p][b],
                         pj_hbm.at[pl.ds(base + c * CH, CH)], wsem)
        pltpu.async_copy(isets[p][b],
                         pi_hbm.at[pl.ds(base + c * CH, CH)], wsem)

    def drain_writes(p):
      for b in range(NB):
        pltpu.make_async_copy(jsets[p][b], pj_hbm.at[pl.ds(0, CH)],
                              wsem).wait()
        pltpu.make_async_copy(isets[p][b], pj_hbm.at[pl.ds(0, CH)],
                              wsem).wait()

    fire_gathers(0, 0)
    wait_gathers(0)
    fire_gathers(1, 1)
    fire_writes(0, 0)

    def body(k, carry):
      g1 = 2 * k + 1
      wait_gathers(1)
      drain_writes(0)
      fire_gathers(g1 + 1, 0)
      fire_writes(g1, 1)
      g2 = 2 * k + 2
      wait_gathers(0)

      @pl.when(g2 + 1 < ngrp)
      def _():
        drain_writes(1)
        fire_gathers(g2 + 1, 1)

      fire_writes(g2, 0)
      return carry

    lax.fori_loop(0, (ngrp - 1) // 2, body, 0)

    if tail:
      ta = pltpu.async_copy(
          pos_hbm.at[idx_s.at[pl.ds(nfull * CH, tail)]],
          rtj.at[pl.ds(0, tail)], tsem)
      tb = pltpu.async_copy(
          pos_hbm.at[idx_d.at[pl.ds(nfull * CH, tail)]],
          rti.at[pl.ds(0, tail)], tsem)
      ta.wait()
      tb.wait()
      pltpu.sync_copy(rtj.at[pl.ds(0, tail)],
                      pj_hbm.at[pl.ds(base + nfull * CH, tail)])
      pltpu.sync_copy(rti.at[pl.ds(0, tail)],
                      pi_hbm.at[pl.ds(base + nfull * CH, tail)])
    for p in range(NSET):
      drain_writes(p)

  return gk(pos16, src, dst)


NB = 3
NSET = 2


def _sc_gather1(x, src):
  n, d = x.shape
  e = src.shape[0]
  ew = e // NW
  nfull = ew // CH
  tail = ew - nfull * CH
  ngrp = nfull // NB
  assert ngrp * NB == nfull and ngrp % 2 == 1
  mesh = plsc.VectorSubcoreMesh(core_axis_name="c", subcore_axis_name="s")
  nbuf = NSET * NB

  @functools.partial(
      pl.kernel,
      out_type=jax.ShapeDtypeStruct((e, d), _f32),
      mesh=mesh,
      scratch_types=(
          [pltpu.VMEM((ew,), jnp.int32)]
          + [pltpu.VMEM((CH, d), _f32) for _ in range(nbuf)]
          + [pltpu.VMEM((max(tail, 8), d), _f32),
             pltpu.SemaphoreType.DMA,
             pltpu.SemaphoreType.DMA,
             pltpu.SemaphoreType.DMA]
      ),
  )
  def gk(x_hbm, src_hbm, xj_hbm, idx_all, *rest):
    bufs = rest[:nbuf]
    rt, gsem, wsem, tsem = rest[nbuf:]
    wid = lax.axis_index("s") * NC + lax.axis_index("c")
    base = wid * ew
    pltpu.sync_copy(src_hbm.at[pl.ds(base, ew)], idx_all)
    sets = (bufs[:NB], bufs[NB:])

    def fire_gathers(g, sbufs):
      for b in range(NB):
        c = g * NB + b
        pltpu.async_copy(x_hbm.at[idx_all.at[pl.ds(c * CH, CH)]],
                         sbufs[b], gsem)

    def wait_gathers(sbufs):
      for b in range(NB):
        pltpu.make_async_copy(xj_hbm.at[pl.ds(0, CH)], sbufs[b], gsem).wait()

    def fire_writes(g, sbufs):
      for b in range(NB):
        c = g * NB + b
        pltpu.async_copy(sbufs[b], xj_hbm.at[pl.ds(base + c * CH, CH)], wsem)

    def drain_writes(sbufs):
      for b in range(NB):
        pltpu.make_async_copy(sbufs[b], xj_hbm.at[pl.ds(0, CH)], wsem).wait()

    fire_gathers(0, sets[0])
    wait_gathers(sets[0])
    fire_gathers(1, sets[1])
    fire_writes(0, sets[0])

    def body(k, carry):
      g1 = 2 * k + 1
      wait_gathers(sets[1])
      drain_writes(sets[0])
      fire_gathers(g1 + 1, sets[0])
      fire_writes(g1, sets[1])
      g2 = 2 * k + 2
      wait_gathers(sets[0])

      @pl.when(g2 + 1 < ngrp)
      def _():
        drain_writes(sets[1])
        fire_gathers(g2 + 1, sets[1])

      fire_writes(g2, sets[0])
      return carry

    lax.fori_loop(0, (ngrp - 1) // 2, body, 0)

    if tail:
      pltpu.async_copy(
          x_hbm.at[idx_all.at[pl.ds(nfull * CH, tail)]],
          rt.at[pl.ds(0, tail)], tsem).wait()
      pltpu.sync_copy(rt.at[pl.ds(0, tail)],
                      xj_hbm.at[pl.ds(base + nfull * CH, tail)])
    for sbufs in sets:
      drain_writes(sbufs)

  return gk(x, src)


def _sc_scatter1(msg, dst, n, dep=None):
  if dep is None:
    dep = jnp.zeros((8,), jnp.int32)
  e, da = msg.shape
  ew = e // NW
  nfull = ew // CH
  etail = ew - nfull * CH
  nb = NB
  ngrp = nfull // nb
  assert ngrp * nb == nfull and ngrp % 2 == 1
  br = (n // NS) // 8 * 8
  ntail = n - br * NS
  mesh = plsc.VectorSubcoreMesh(core_axis_name="c", subcore_axis_name="s")
  nbuf = NSET * nb

  zeros_a = jnp.zeros((max(br, ntail), da), _f32)

  @functools.partial(
      pl.kernel,
      out_type=jax.ShapeDtypeStruct((NC, n, da), _f32),
      mesh=mesh,
      scratch_types=(
          [pltpu.VMEM((CH,), jnp.int32) for _ in range(nbuf)]
          + [pltpu.VMEM((CH, da), _f32) for _ in range(nbuf)]
          + [pltpu.VMEM((max(etail, 8),), jnp.int32),
             pltpu.VMEM((max(etail, 8), da), _f32),
             pltpu.VMEM_SHARED((n, da), _f32),
             pltpu.SemaphoreType.DMA,
             pltpu.SemaphoreType.DMA,
             pltpu.SemaphoreType.DMA]
      ),
      compiler_params=pltpu.CompilerParams(use_tc_tiling_on_sc=False),
  )
  def sk(msg_hbm, dst_hbm, za_hbm, dep_hbm, pa_hbm, *rest):
    del dep_hbm
    ibufs = rest[:nbuf]
    mbufs = rest[nbuf:2 * nbuf]
    ti, tm, acc_a, lsem, ssem, tsem = rest[2 * nbuf:]
    cid = lax.axis_index("c")
    sid = lax.axis_index("s")
    wid = sid * NC + cid
    base = wid * ew
    r0 = sid * br
    pltpu.sync_copy(za_hbm.at[pl.ds(0, br)], acc_a.at[pl.ds(r0, br)])

    @pl.when(sid == NS - 1)
    def _():
      pltpu.sync_copy(za_hbm.at[pl.ds(0, ntail)],
                      acc_a.at[pl.ds(NS * br, ntail)])

    plsc.subcore_barrier()

    isets = (ibufs[:nb], ibufs[nb:])
    msets = (mbufs[:nb], mbufs[nb:])

    def fire_loads(g, p):
      for b in range(nb):
        c = g * NB + b
        pltpu.async_copy(dst_hbm.at[pl.ds(base + c * CH, CH)],
                         isets[p][b], lsem)
        pltpu.async_copy(msg_hbm.at[pl.ds(base + c * CH, CH)],
                         msets[p][b], lsem)

    def wait_loads(p):
      for b in range(nb):
        pltpu.make_async_copy(dst_hbm.at[pl.ds(0, CH)], isets[p][b],
                              lsem).wait()
        pltpu.make_async_copy(msg_hbm.at[pl.ds(0, CH)], msets[p][b],
                              lsem).wait()

    def fire_adds(p):
      for b in range(nb):
        pltpu.async_copy(msets[p][b], acc_a.at[isets[p][b]], ssem, add=True)

    def drain_adds(p):
      for b in range(nb):
        pltpu.make_async_copy(msets[p][b], acc_a.at[pl.ds(0, CH)],
                              ssem).wait()

    fire_loads(0, 0)
    wait_loads(0)
    fire_loads(1, 1)
    fire_adds(0)

    def body(k, carry):
      wait_loads(1)
      drain_adds(0)
      fire_loads(2 * k + 2, 0)
      fire_adds(1)
      wait_loads(0)

      @pl.when(2 * k + 3 < ngrp)
      def _():
        drain_adds(1)
        fire_loads(2 * k + 3, 1)

      fire_adds(0)
      return carry

    lax.fori_loop(0, (ngrp - 1) // 2, body, 0)

    if etail:
      ta = pltpu.async_copy(dst_hbm.at[pl.ds(base + nfull * CH, etail)],
                            ti.at[pl.ds(0, etail)], tsem)
      tb = pltpu.async_copy(msg_hbm.at[pl.ds(base + nfull * CH, etail)],
                            tm.at[pl.ds(0, etail)], tsem)
      ta.wait()
      tb.wait()
      pltpu.sync_copy(tm.at[pl.ds(0, etail)], acc_a.at[ti], add=True)
    for p in range(NSET):
      drain_adds(p)
    plsc.subcore_barrier()
    pltpu.sync_copy(acc_a.at[pl.ds(r0, br)], pa_hbm.at[cid, pl.ds(r0, br)])

    @pl.when(sid == NS - 1)
    def _():
      pltpu.sync_copy(acc_a.at[pl.ds(NS * br, ntail)],
                      pa_hbm.at[cid, pl.ds(NS * br, ntail)])

  return sk(msg, dst, zeros_a, dep)


def _sc_scatter_cols(msg2, dst, n, dep=None):
  if dep is None:
    dep = jnp.zeros((8,), jnp.int32)
  e, da = msg2.shape
  dh = da // NC
  ept = e // NS
  nfull = ept // CH
  etail = ept - nfull * CH
  nb = 2
  ngrp = nfull // nb
  assert ngrp * nb == nfull and ngrp % 2 == 1
  br = (n // NS) // 8 * 8
  ntail = n - br * NS
  mesh = plsc.VectorSubcoreMesh(core_axis_name="c", subcore_axis_name="s")
  nbuf = NSET * nb

  zeros_a = jnp.zeros((max(br, ntail), dh), _f32)

  @functools.partial(
      pl.kernel,
      out_type=jax.ShapeDtypeStruct((NC, n, dh), _f32),
      mesh=mesh,
      scratch_types=(
          [pltpu.VMEM((CH,), jnp.int32) for _ in range(nbuf)]
          + [pltpu.VMEM((CH, dh), _f32) for _ in range(nbuf)]
          + [pltpu.VMEM((max(etail, 8),), jnp.int32),
             pltpu.VMEM((max(etail, 8), dh), _f32),
             pltpu.VMEM_SHARED((n, dh), _f32),
             pltpu.SemaphoreType.DMA,
             pltpu.SemaphoreType.DMA,
             pltpu.SemaphoreType.DMA]
      ),
      compiler_params=pltpu.CompilerParams(use_tc_tiling_on_sc=False),
  )
  def sk(msg_hbm, dst_hbm, za_hbm, dep_hbm, pa_hbm, *rest):
    del dep_hbm
    ibufs = rest[:nbuf]
    mbufs = rest[nbuf:2 * nbuf]
    ti, tm, acc_a, lsem, ssem, tsem = rest[2 * nbuf:]
    cid = lax.axis_index("c")
    sid = lax.axis_index("s")
    base = sid * ept
    r0 = sid * br
    pltpu.sync_copy(za_hbm.at[pl.ds(0, br)], acc_a.at[pl.ds(r0, br)])

    @pl.when(sid == NS - 1)
    def _():
      pltpu.sync_copy(za_hbm.at[pl.ds(0, ntail)],
                      acc_a.at[pl.ds(NS * br, ntail)])

    plsc.subcore_barrier()

    isets = (ibufs[:nb], ibufs[nb:])
    msets = (mbufs[:nb], mbufs[nb:])

    def fire_loads(g, p):
      for b in range(nb):
        c = g * nb + b
        pltpu.async_copy(dst_hbm.at[pl.ds(base + c * CH, CH)],
                         isets[p][b], lsem)
        pltpu.async_copy(
            msg_hbm.at[pl.ds(base + c * CH, CH), pl.ds(cid * dh, dh)],
            msets[p][b], lsem)

    def wait_loads(p):
      for b in range(nb):
        pltpu.make_async_copy(dst_hbm.at[pl.ds(0, CH)], isets[p][b],
                              lsem).wait()
        pltpu.make_async_copy(msg_hbm.at[pl.ds(0, CH), pl.ds(0, dh)],
                              msets[p][b], lsem).wait()

    def fire_adds(p):
      for b in range(nb):
        pltpu.async_copy(msets[p][b], acc_a.at[isets[p][b]], ssem, add=True)

    def drain_adds(p):
      for b in range(nb):
        pltpu.make_async_copy(msets[p][b], acc_a.at[pl.ds(0, CH)],
                              ssem).wait()

    fire_loads(0, 0)
    wait_loads(0)
    fire_loads(1, 1)
    fire_adds(0)

    def body(k, carry):
      wait_loads(1)
      drain_adds(0)
      fire_loads(2 * k + 2, 0)
      fire_adds(1)
      wait_loads(0)

      @pl.when(2 * k + 3 < ngrp)
      def _():
        drain_adds(1)
        fire_loads(2 * k + 3, 1)

      fire_adds(0)
      return carry

    lax.fori_loop(0, (ngrp - 1) // 2, body, 0)

    if etail:
      ta = pltpu.async_copy(dst_hbm.at[pl.ds(base + nfull * CH, etail)],
                            ti.at[pl.ds(0, etail)], tsem)
      tb = pltpu.async_copy(
          msg_hbm.at[pl.ds(base + nfull * CH, etail), pl.ds(cid * dh, dh)],
          tm.at[pl.ds(0, etail)], tsem)
      ta.wait()
      tb.wait()
      pltpu.sync_copy(tm.at[pl.ds(0, etail)], acc_a.at[ti], add=True)
    for p in range(NSET):
      drain_adds(p)
    plsc.subcore_barrier()
    pltpu.sync_copy(acc_a.at[pl.ds(r0, br)], pa_hbm.at[cid, pl.ds(r0, br)])

    @pl.when(sid == NS - 1)
    def _():
      pltpu.sync_copy(acc_a.at[pl.ds(NS * br, ntail)],
                      pa_hbm.at[cid, pl.ds(NS * br, ntail)])

  return sk(msg2, dst, zeros_a, dep)


def _sc_scatter(msg_a, msg_b, dst, n):
  pa = _sc_scatter_cols(msg_a, dst, n)
  pb = _sc_scatter1(msg_b, dst, n, dep=pa)
  return pa, pb


def _edge_body(xj_ref, pi_ref, pj_ref, wpi_ref, wpj_ref, w9_ref, ppb_ref,
               awa_ref, awb_ref, ab_ref, oa_ref, ob_ref):
  xj = xj_ref[...]
  pi = pi_ref[...]
  pj = pj_ref[...]
  v = pi - pj
  dij = jnp.sqrt(jnp.sum(v * v, axis=1, keepdims=True))
  r = pi @ wpi_ref[...] + pj @ wpj_ref[...] + dij * w9_ref[...] + ppb_ref[...]
  r = jnp.maximum(r, 0.0)
  g = xj @ awa_ref[...] + r @ awb_ref[...] + ab_ref[...]
  g = jnp.maximum(g, 0.0)
  m = jnp.max(g, axis=1, keepdims=True)
  eg = jnp.exp(g - m)
  s = eg / jnp.sum(eg, axis=1, keepdims=True)
  da = xj.shape[1]
  oa_ref[...] = s[:, :da] * xj
  ob_ref[...] = s[:, da:] * r


def _tc_edge(xj, pi16, pj16, wpi, wpj, w9, ppb, awa, awb, ab, block=1000):
  e, d = xj.shape
  dp = wpi.shape[1]
  df = d + dp
  dh = d // 2
  grid = e // block
  full = lambda i: (0, 0)
  return pl.pallas_call(
      _edge_body,
      grid=(grid,),
      in_specs=[
          pl.BlockSpec((block, d), lambda i: (i, 0)),
          pl.BlockSpec((block, 16), lambda i: (i, 0)),
          pl.BlockSpec((block, 16), lambda i: (i, 0)),
          pl.BlockSpec((16, dp), full),
          pl.BlockSpec((16, dp), full),
          pl.BlockSpec((1, dp), full),
          pl.BlockSpec((1, dp), full),
          pl.BlockSpec((d, df), full),
          pl.BlockSpec((dp, df), full),
          pl.BlockSpec((1, df), full),
      ],
      out_specs=[
          pl.BlockSpec((block, d), lambda i: (i, 0)),
          pl.BlockSpec((block, dp), lambda i: (i, 0)),
      ],
      out_shape=[
          jax.ShapeDtypeStruct((e, d), _f32),
          jax.ShapeDtypeStruct((e, dp), _f32),
      ],
  )(xj, pi16, pj16, wpi, wpj, w9, ppb, awa, awb, ab)


def _global_body(pa_ref, pb_ref, gwa0_ref, gwa1_ref, gwb_ref, gb_ref, o_ref):
  b = pb_ref[0] + pb_ref[1]
  o_ref[...] = jnp.maximum(
      pa_ref[0] @ gwa0_ref[...] + pa_ref[1] @ gwa1_ref[...]
      + b @ gwb_ref[...] + gb_ref[...], 0.0)


def _tc_global(pa, pb, gwa0, gwa1, gwb, gb, block=1000):
  _, n, dh = pa.shape
  db = pb.shape[2]
  dout = gwa0.shape[1]
  grid = n // block
  full = lambda i: (0, 0)
  return pl.pallas_call(
      _global_body,
      grid=(grid,),
      in_specs=[
          pl.BlockSpec((NC, block, dh), lambda i: (0, i, 0)),
          pl.BlockSpec((NC, block, db), lambda i: (0, i, 0)),
          pl.BlockSpec((dh, dout), full),
          pl.BlockSpec((dh, dout), full),
          pl.BlockSpec((db, dout), full),
          pl.BlockSpec((1, dout), full),
      ],
      out_specs=pl.BlockSpec((block, dout), lambda i: (i, 0)),
      out_shape=jax.ShapeDtypeStruct((n, dout), _f32),
  )(pa, pb, gwa0, gwa1, gwb, gb)


def _global_res_body(pa_ref, pb_ref, x_ref, gwa0_ref, gwa1_ref, gwb_ref,
                     gb_ref, scw_ref, scb_ref, o_ref):
  b = pb_ref[0] + pb_ref[1]
  h = (pa_ref[0] @ gwa0_ref[...] + pa_ref[1] @ gwa1_ref[...]
       + b @ gwb_ref[...] + gb_ref[...])
  h = jnp.maximum(h, 0.0)
  sc = x_ref[...] @ scw_ref[...] + scb_ref[...]
  o_ref[...] = jnp.maximum(h + sc, 0.0)


def _tc_global_res(pa, pb, x, gwa0, gwa1, gwb, gb, scw, scb, block=1000):
  _, n, dh = pa.shape
  db = pb.shape[2]
  d = x.shape[1]
  dout = gwa0.shape[1]
  grid = n // block
  full = lambda i: (0, 0)
  return pl.pallas_call(
      _global_res_body,
      grid=(grid,),
      in_specs=[
          pl.BlockSpec((NC, block, dh), lambda i: (0, i, 0)),
          pl.BlockSpec((NC, block, db), lambda i: (0, i, 0)),
          pl.BlockSpec((block, d), lambda i: (i, 0)),
          pl.BlockSpec((dh, dout), full),
          pl.BlockSpec((dh, dout), full),
          pl.BlockSpec((db, dout), full),
          pl.BlockSpec((1, dout), full),
          pl.BlockSpec((d, dout), full),
          pl.BlockSpec((1, dout), full),
      ],
      out_specs=pl.BlockSpec((block, dout), lambda i: (i, 0)),
      out_shape=jax.ShapeDtypeStruct((n, dout), _f32),
  )(pa, pb, x, gwa0, gwa1, gwb, gb, scw, scb)


def _prep_pp(ppW):
  wpi = jnp.zeros((16, ppW.shape[1]), _f32).at[:3].set(ppW[0:3] + ppW[6:9])
  wpj = jnp.zeros((16, ppW.shape[1]), _f32).at[:3].set(ppW[3:6] - ppW[6:9])
  w9 = ppW[9:10]
  return wpi, wpj, w9


def kernel(x, pos, edge_index, ppW1, ppb1, aW1, ab1, gW1, gb1,
           ppW2, ppb2, aW2, ab2, gW2, gb2, scW, scb):
  n, d = x.shape
  src = edge_index[0]
  dst = edge_index[1]
  pos16 = jnp.zeros((n, 16), _f32).at[:, :3].set(pos)

  pj16, pi16 = _sc_gather_pos(pos16, src, dst)
  xj = _sc_gather1(x, src)

  dh = d // 2
  wpi1, wpj1, w91 = _prep_pp(ppW1)
  ma1, mb1 = _tc_edge(xj, pi16, pj16, wpi1, wpj1, w91, ppb1[None, :],
                      aW1[:d], aW1[d:], ab1[None, :])
  pa1, pb1 = _sc_scatter(ma1, mb1, dst, n)
  h1 = _tc_global(pa1, pb1, gW1[:dh], gW1[dh:d], gW1[d:], gb1[None, :])

  hj = _sc_gather1(h1, src)
  wpi2, wpj2, w92 = _prep_pp(ppW2)
  ma2, mb2 = _tc_edge(hj, pi16, pj16, wpi2, wpj2, w92, ppb2[None, :],
                      aW2[:d], aW2[d:], ab2[None, :])
  pa2, pb2 = _sc_scatter(ma2, mb2, dst, n)
  out = _tc_global_res(pa2, pb2, x, gW2[:dh], gW2[dh:d], gW2[d:],
                       gb2[None, :], scW, scb[None, :])
  return out


# speedup vs baseline: 1.0671x; 1.0015x over previous
import functools

import jax
import jax.numpy as jnp
from jax import lax
from jax.experimental import pallas as pl
from jax.experimental.pallas import tpu as pltpu
from jax.experimental.pallas import tpu_sc as plsc

NC = 2
NS = 16
NW = NC * NS
CH = 128

_f32 = jnp.float32


def _sc_gather_pos(pos16, src, dst):
  e = src.shape[0]
  ew = e // NW
  nfull = ew // CH
  tail = ew - nfull * CH
  ngrp = nfull // NB
  assert ngrp * NB == nfull and ngrp % 2 == 1
  mesh = plsc.VectorSubcoreMesh(core_axis_name="c", subcore_axis_name="s")
  nbuf = NSET * NB

  @functools.partial(
      pl.kernel,
      out_type=[
          jax.ShapeDtypeStruct((e, 16), _f32),
          jax.ShapeDtypeStruct((e, 16), _f32),
      ],
      mesh=mesh,
      scratch_types=(
          [pltpu.VMEM((ew,), jnp.int32), pltpu.VMEM((ew,), jnp.int32)]
          + [pltpu.VMEM((CH, 16), _f32) for _ in range(2 * nbuf)]
          + [pltpu.VMEM((max(tail, 8), 16), _f32),
             pltpu.VMEM((max(tail, 8), 16), _f32),
             pltpu.SemaphoreType.DMA,
             pltpu.SemaphoreType.DMA,
             pltpu.SemaphoreType.DMA]
      ),
      compiler_params=pltpu.CompilerParams(use_tc_tiling_on_sc=False),
  )
  def gk(pos_hbm, src_hbm, dst_hbm, pj_hbm, pi_hbm, idx_s, idx_d, *rest):
    jbufs = rest[:nbuf]
    ibufs = rest[nbuf:2 * nbuf]
    rtj, rti, gsem, wsem, tsem = rest[2 * nbuf:]
    wid = lax.axis_index("s") * NC + lax.axis_index("c")
    base = wid * ew
    pltpu.sync_copy(src_hbm.at[pl.ds(base, ew)], idx_s)
    pltpu.sync_copy(dst_hbm.at[pl.ds(base, ew)], idx_d)
    jsets = (jbufs[:NB], jbufs[NB:])
    isets = (ibufs[:NB], ibufs[NB:])

    def fire_gathers(g, p):
      for b in range(NB):
        c = g * NB + b
        pltpu.async_copy(pos_hbm.at[idx_s.at[pl.ds(c * CH, CH)]],
                         jsets[p][b], gsem)
        pltpu.async_copy(pos_hbm.at[idx_d.at[pl.ds(c * CH, CH)]],
                         isets[p][b], gsem)

    def wait_gathers(p):
      for b in range(NB):
        pltpu.make_async_copy(pj_hbm.at[pl.ds(0, CH)], jsets[p][b],
                              gsem).wait()
        pltpu.make_async_copy(pj_hbm.at[pl.ds(0, CH)], isets[p][b],
                              gsem).wait()

    def fire_writes(g, p):
      for b in range(NB):
        c = g * NB + b
        pltpu.async_copy(jsets[p][b],
                         pj_hbm.at[pl.ds(base + c * CH, CH)], wsem)
        pltpu.async_copy(isets[p][b],
                         pi_hbm.at[pl.ds(base + c * CH, CH)], wsem)

    def drain_writes(p):
      for b in range(NB):
        pltpu.make_async_copy(jsets[p][b], pj_hbm.at[pl.ds(0, CH)],
                              wsem).wait()
        pltpu.make_async_copy(isets[p][b], pj_hbm.at[pl.ds(0, CH)],
                              wsem).wait()

    fire_gathers(0, 0)
    wait_gathers(0)
    fire_gathers(1, 1)
    fire_writes(0, 0)

    def body(k, carry):
      g1 = 2 * k + 1
      wait_gathers(1)
      drain_writes(0)
      fire_gathers(g1 + 1, 0)
      fire_writes(g1, 1)
      g2 = 2 * k + 2
      wait_gathers(0)

      @pl.when(g2 + 1 < ngrp)
      def _():
        drain_writes(1)
        fire_gathers(g2 + 1, 1)

      fire_writes(g2, 0)
      return carry

    lax.fori_loop(0, (ngrp - 1) // 2, body, 0)

    if tail:
      ta = pltpu.async_copy(
          pos_hbm.at[idx_s.at[pl.ds(nfull * CH, tail)]],
          rtj.at[pl.ds(0, tail)], tsem)
      tb = pltpu.async_copy(
          pos_hbm.at[idx_d.at[pl.ds(nfull * CH, tail)]],
          rti.at[pl.ds(0, tail)], tsem)
      ta.wait()
      tb.wait()
      pltpu.sync_copy(rtj.at[pl.ds(0, tail)],
                      pj_hbm.at[pl.ds(base + nfull * CH, tail)])
      pltpu.sync_copy(rti.at[pl.ds(0, tail)],
                      pi_hbm.at[pl.ds(base + nfull * CH, tail)])
    for p in range(NSET):
      drain_writes(p)

  return gk(pos16, src, dst)


NB = 3
NSET = 2


def _sc_gather1(x, src):
  n, d = x.shape
  e = src.shape[0]
  ew = e // NW
  nfull = ew // CH
  tail = ew - nfull * CH
  ngrp = nfull // NB
  assert ngrp * NB == nfull and ngrp % 2 == 1
  mesh = plsc.VectorSubcoreMesh(core_axis_name="c", subcore_axis_name="s")
  nbuf = NSET * NB

  @functools.partial(
      pl.kernel,
      out_type=jax.ShapeDtypeStruct((e, d), _f32),
      mesh=mesh,
      scratch_types=(
          [pltpu.VMEM((ew,), jnp.int32)]
          + [pltpu.VMEM((CH, d), _f32) for _ in range(nbuf)]
          + [pltpu.VMEM((max(tail, 8), d), _f32),
             pltpu.SemaphoreType.DMA,
             pltpu.SemaphoreType.DMA,
             pltpu.SemaphoreType.DMA]
      ),
  )
  def gk(x_hbm, src_hbm, xj_hbm, idx_all, *rest):
    bufs = rest[:nbuf]
    rt, gsem, wsem, tsem = rest[nbuf:]
    wid = lax.axis_index("s") * NC + lax.axis_index("c")
    base = wid * ew
    pltpu.sync_copy(src_hbm.at[pl.ds(base, ew)], idx_all)
    sets = (bufs[:NB], bufs[NB:])

    def fire_gathers(g, sbufs):
      for b in range(NB):
        c = g * NB + b
        pltpu.async_copy(x_hbm.at[idx_all.at[pl.ds(c * CH, CH)]],
                         sbufs[b], gsem)

    def wait_gathers(sbufs):
      for b in range(NB):
        pltpu.make_async_copy(xj_hbm.at[pl.ds(0, CH)], sbufs[b], gsem).wait()

    def fire_writes(g, sbufs):
      for b in range(NB):
        c = g * NB + b
        pltpu.async_copy(sbufs[b], xj_hbm.at[pl.ds(base + c * CH, CH)], wsem)

    def drain_writes(sbufs):
      for b in range(NB):
        pltpu.make_async_copy(sbufs[b], xj_hbm.at[pl.ds(0, CH)], wsem).wait()

    fire_gathers(0, sets[0])
    wait_gathers(sets[0])
    fire_gathers(1, sets[1])
    fire_writes(0, sets[0])

    def body(k, carry):
      g1 = 2 * k + 1
      wait_gathers(sets[1])
      drain_writes(sets[0])
      fire_gathers(g1 + 1, sets[0])
      fire_writes(g1, sets[1])
      g2 = 2 * k + 2
      wait_gathers(sets[0])

      @pl.when(g2 + 1 < ngrp)
      def _():
        drain_writes(sets[1])
        fire_gathers(g2 + 1, sets[1])

      fire_writes(g2, sets[0])
      return carry

    lax.fori_loop(0, (ngrp - 1) // 2, body, 0)

    if tail:
      pltpu.async_copy(
          x_hbm.at[idx_all.at[pl.ds(nfull * CH, tail)]],
          rt.at[pl.ds(0, tail)], tsem).wait()
      pltpu.sync_copy(rt.at[pl.ds(0, tail)],
                      xj_hbm.at[pl.ds(base + nfull * CH, tail)])
    for sbufs in sets:
      drain_writes(sbufs)

  return gk(x, src)


def _sc_scatter1(msg, dst, n, dep=None):
  if dep is None:
    dep = jnp.zeros((8,), jnp.int32)
  e, da = msg.shape
  ew = e // NW
  nfull = ew // CH
  etail = ew - nfull * CH
  nb = NB
  ngrp = nfull // nb
  assert ngrp * nb == nfull and ngrp % 2 == 1
  br = (n // NS) // 8 * 8
  ntail = n - br * NS
  mesh = plsc.VectorSubcoreMesh(core_axis_name="c", subcore_axis_name="s")
  nbuf = NSET * nb

  zeros_a = jnp.zeros((max(br, ntail), da), _f32)

  @functools.partial(
      pl.kernel,
      out_type=jax.ShapeDtypeStruct((NC, n, da), _f32),
      mesh=mesh,
      scratch_types=(
          [pltpu.VMEM((CH,), jnp.int32) for _ in range(nbuf)]
          + [pltpu.VMEM((CH, da), _f32) for _ in range(nbuf)]
          + [pltpu.VMEM((max(etail, 8),), jnp.int32),
             pltpu.VMEM((max(etail, 8), da), _f32),
             pltpu.VMEM_SHARED((n, da), _f32),
             pltpu.SemaphoreType.DMA,
             pltpu.SemaphoreType.DMA,
             pltpu.SemaphoreType.DMA]
      ),
      compiler_params=pltpu.CompilerParams(use_tc_tiling_on_sc=False),
  )
  def sk(msg_hbm, dst_hbm, za_hbm, dep_hbm, pa_hbm, *rest):
    del dep_hbm
    ibufs = rest[:nbuf]
    mbufs = rest[nbuf:2 * nbuf]
    ti, tm, acc_a, lsem, ssem, tsem = rest[2 * nbuf:]
    cid = lax.axis_index("c")
    sid = lax.axis_index("s")
    wid = sid * NC + cid
    base = wid * ew
    r0 = sid * br
    pltpu.sync_copy(za_hbm.at[pl.ds(0, br)], acc_a.at[pl.ds(r0, br)])

    @pl.when(sid == NS - 1)
    def _():
      pltpu.sync_copy(za_hbm.at[pl.ds(0, ntail)],
                      acc_a.at[pl.ds(NS * br, ntail)])

    plsc.subcore_barrier()

    isets = (ibufs[:nb], ibufs[nb:])
    msets = (mbufs[:nb], mbufs[nb:])

    def fire_loads(g, p):
      for b in range(nb):
        c = g * NB + b
        pltpu.async_copy(dst_hbm.at[pl.ds(base + c * CH, CH)],
                         isets[p][b], lsem)
        pltpu.async_copy(msg_hbm.at[pl.ds(base + c * CH, CH)],
                         msets[p][b], lsem)

    def wait_loads(p):
      for b in range(nb):
        pltpu.make_async_copy(dst_hbm.at[pl.ds(0, CH)], isets[p][b],
                              lsem).wait()
        pltpu.make_async_copy(msg_hbm.at[pl.ds(0, CH)], msets[p][b],
                              lsem).wait()

    def fire_adds(p):
      for b in range(nb):
        pltpu.async_copy(msets[p][b], acc_a.at[isets[p][b]], ssem, add=True)

    def drain_adds(p):
      for b in range(nb):
        pltpu.make_async_copy(msets[p][b], acc_a.at[pl.ds(0, CH)],
                              ssem).wait()

    fire_loads(0, 0)
    wait_loads(0)
    fire_loads(1, 1)
    fire_adds(0)

    def body(k, carry):
      wait_loads(1)
      drain_adds(0)
      fire_loads(2 * k + 2, 0)
      fire_adds(1)
      wait_loads(0)

      @pl.when(2 * k + 3 < ngrp)
      def _():
        drain_adds(1)
        fire_loads(2 * k + 3, 1)

      fire_adds(0)
      return carry

    lax.fori_loop(0, (ngrp - 1) // 2, body, 0)

    if etail:
      ta = pltpu.async_copy(dst_hbm.at[pl.ds(base + nfull * CH, etail)],
                            ti.at[pl.ds(0, etail)], tsem)
      tb = pltpu.async_copy(msg_hbm.at[pl.ds(base + nfull * CH, etail)],
                            tm.at[pl.ds(0, etail)], tsem)
      ta.wait()
      tb.wait()
      pltpu.sync_copy(tm.at[pl.ds(0, etail)], acc_a.at[ti], add=True)
    for p in range(NSET):
      drain_adds(p)
    plsc.subcore_barrier()
    pltpu.sync_copy(acc_a.at[pl.ds(r0, br)], pa_hbm.at[cid, pl.ds(r0, br)])

    @pl.when(sid == NS - 1)
    def _():
      pltpu.sync_copy(acc_a.at[pl.ds(NS * br, ntail)],
                      pa_hbm.at[cid, pl.ds(NS * br, ntail)])

  return sk(msg, dst, zeros_a, dep)


def _sc_scatter_cols(msg2, dst, n, dep=None):
  if dep is None:
    dep = jnp.zeros((8,), jnp.int32)
  e, da = msg2.shape
  dh = da // NC
  ept = e // NS
  nfull = ept // CH
  etail = ept - nfull * CH
  nb = 2
  ngrp = nfull // nb
  assert ngrp * nb == nfull and ngrp % 2 == 1
  br = (n // NS) // 8 * 8
  ntail = n - br * NS
  mesh = plsc.VectorSubcoreMesh(core_axis_name="c", subcore_axis_name="s")
  nbuf = NSET * nb

  zeros_a = jnp.zeros((max(br, ntail), dh), _f32)

  @functools.partial(
      pl.kernel,
      out_type=jax.ShapeDtypeStruct((NC, n, dh), _f32),
      mesh=mesh,
      scratch_types=(
          [pltpu.VMEM((CH,), jnp.int32) for _ in range(nbuf)]
          + [pltpu.VMEM((CH, dh), _f32) for _ in range(nbuf)]
          + [pltpu.VMEM((max(etail, 8),), jnp.int32),
             pltpu.VMEM((max(etail, 8), dh), _f32),
             pltpu.VMEM_SHARED((n, dh), _f32),
             pltpu.SemaphoreType.DMA,
             pltpu.SemaphoreType.DMA,
             pltpu.SemaphoreType.DMA]
      ),
      compiler_params=pltpu.CompilerParams(use_tc_tiling_on_sc=False),
  )
  def sk(msg_hbm, dst_hbm, za_hbm, dep_hbm, pa_hbm, *rest):
    del dep_hbm
    ibufs = rest[:nbuf]
    mbufs = rest[nbuf:2 * nbuf]
    ti, tm, acc_a, lsem, ssem, tsem = rest[2 * nbuf:]
    cid = lax.axis_index("c")
    sid = lax.axis_index("s")
    base = sid * ept
    r0 = sid * br
    pltpu.sync_copy(za_hbm.at[pl.ds(0, br)], acc_a.at[pl.ds(r0, br)])

    @pl.when(sid == NS - 1)
    def _():
      pltpu.sync_copy(za_hbm.at[pl.ds(0, ntail)],
                      acc_a.at[pl.ds(NS * br, ntail)])

    plsc.subcore_barrier()

    isets = (ibufs[:nb], ibufs[nb:])
    msets = (mbufs[:nb], mbufs[nb:])

    def fire_loads(g, p):
      for b in range(nb):
        c = g * nb + b
        pltpu.async_copy(dst_hbm.at[pl.ds(base + c * CH, CH)],
                         isets[p][b], lsem)
        pltpu.async_copy(
            msg_hbm.at[pl.ds(base + c * CH, CH), pl.ds(cid * dh, dh)],
            msets[p][b], lsem)

    def wait_loads(p):
      for b in range(nb):
        pltpu.make_async_copy(dst_hbm.at[pl.ds(0, CH)], isets[p][b],
                              lsem).wait()
        pltpu.make_async_copy(msg_hbm.at[pl.ds(0, CH), pl.ds(0, dh)],
                              msets[p][b], lsem).wait()

    def fire_adds(p):
      for b in range(nb):
        pltpu.async_copy(msets[p][b], acc_a.at[isets[p][b]], ssem, add=True)

    def drain_adds(p):
      for b in range(nb):
        pltpu.make_async_copy(msets[p][b], acc_a.at[pl.ds(0, CH)],
                              ssem).wait()

    fire_loads(0, 0)
    wait_loads(0)
    fire_loads(1, 1)
    fire_adds(0)

    def body(k, carry):
      wait_loads(1)
      drain_adds(0)
      fire_loads(2 * k + 2, 0)
      fire_adds(1)
      wait_loads(0)

      @pl.when(2 * k + 3 < ngrp)
      def _():
        drain_adds(1)
        fire_loads(2 * k + 3, 1)

      fire_adds(0)
      return carry

    lax.fori_loop(0, (ngrp - 1) // 2, body, 0)

    if etail:
      ta = pltpu.async_copy(dst_hbm.at[pl.ds(base + nfull * CH, etail)],
                            ti.at[pl.ds(0, etail)], tsem)
      tb = pltpu.async_copy(
          msg_hbm.at[pl.ds(base + nfull * CH, etail), pl.ds(cid * dh, dh)],
          tm.at[pl.ds(0, etail)], tsem)
      ta.wait()
      tb.wait()
      pltpu.sync_copy(tm.at[pl.ds(0, etail)], acc_a.at[ti], add=True)
    for p in range(NSET):
      drain_adds(p)
    plsc.subcore_barrier()
    pltpu.sync_copy(acc_a.at[pl.ds(r0, br)], pa_hbm.at[cid, pl.ds(r0, br)])

    @pl.when(sid == NS - 1)
    def _():
      pltpu.sync_copy(acc_a.at[pl.ds(NS * br, ntail)],
                      pa_hbm.at[cid, pl.ds(NS * br, ntail)])

  return sk(msg2, dst, zeros_a, dep)


def _sc_scatter(msg_a, msg_b, dst, n):
  pa = _sc_scatter_cols(msg_a, dst, n)
  pb = _sc_scatter1(msg_b, dst, n, dep=pa)
  return pa, pb


def _edge_body(xj_ref, pi_ref, pj_ref, wpi_ref, wpj_ref, w9_ref, ppb_ref,
               awa_ref, awb_ref, ab_ref, oa_ref, ob_ref):
  xj = xj_ref[...]
  pi = pi_ref[...]
  pj = pj_ref[...]
  v = pi - pj
  dij = jnp.sqrt(jnp.sum(v * v, axis=1, keepdims=True))
  r = pi @ wpi_ref[...] + pj @ wpj_ref[...] + dij * w9_ref[...] + ppb_ref[...]
  r = jnp.maximum(r, 0.0)
  bf = jnp.bfloat16
  g = (jnp.dot(xj.astype(bf), awa_ref[...].astype(bf),
               preferred_element_type=_f32)
       + jnp.dot(r.astype(bf), awb_ref[...].astype(bf),
                 preferred_element_type=_f32)
       + ab_ref[...])
  g = jnp.maximum(g, 0.0)
  m = jnp.max(g, axis=1, keepdims=True)
  eg = jnp.exp(g - m)
  s = eg / jnp.sum(eg, axis=1, keepdims=True)
  da = xj.shape[1]
  oa_ref[...] = s[:, :da] * xj
  ob_ref[...] = s[:, da:] * r


def _tc_edge(xj, pi16, pj16, wpi, wpj, w9, ppb, awa, awb, ab, block=1000):
  e, d = xj.shape
  dp = wpi.shape[1]
  df = d + dp
  dh = d // 2
  grid = e // block
  full = lambda i: (0, 0)
  return pl.pallas_call(
      _edge_body,
      grid=(grid,),
      in_specs=[
          pl.BlockSpec((block, d), lambda i: (i, 0)),
          pl.BlockSpec((block, 16), lambda i: (i, 0)),
          pl.BlockSpec((block, 16), lambda i: (i, 0)),
          pl.BlockSpec((16, dp), full),
          pl.BlockSpec((16, dp), full),
          pl.BlockSpec((1, dp), full),
          pl.BlockSpec((1, dp), full),
          pl.BlockSpec((d, df), full),
          pl.BlockSpec((dp, df), full),
          pl.BlockSpec((1, df), full),
      ],
      out_specs=[
          pl.BlockSpec((block, d), lambda i: (i, 0)),
          pl.BlockSpec((block, dp), lambda i: (i, 0)),
      ],
      out_shape=[
          jax.ShapeDtypeStruct((e, d), _f32),
          jax.ShapeDtypeStruct((e, dp), _f32),
      ],
  )(xj, pi16, pj16, wpi, wpj, w9, ppb, awa, awb, ab)


def _global_body(pa_ref, pb_ref, gwa0_ref, gwa1_ref, gwb_ref, gb_ref, o_ref):
  b = pb_ref[0] + pb_ref[1]
  o_ref[...] = jnp.maximum(
      pa_ref[0] @ gwa0_ref[...] + pa_ref[1] @ gwa1_ref[...]
      + b @ gwb_ref[...] + gb_ref[...], 0.0)


def _tc_global(pa, pb, gwa0, gwa1, gwb, gb, block=1000):
  _, n, dh = pa.shape
  db = pb.shape[2]
  dout = gwa0.shape[1]
  grid = n // block
  full = lambda i: (0, 0)
  return pl.pallas_call(
      _global_body,
      grid=(grid,),
      in_specs=[
          pl.BlockSpec((NC, block, dh), lambda i: (0, i, 0)),
          pl.BlockSpec((NC, block, db), lambda i: (0, i, 0)),
          pl.BlockSpec((dh, dout), full),
          pl.BlockSpec((dh, dout), full),
          pl.BlockSpec((db, dout), full),
          pl.BlockSpec((1, dout), full),
      ],
      out_specs=pl.BlockSpec((block, dout), lambda i: (i, 0)),
      out_shape=jax.ShapeDtypeStruct((n, dout), _f32),
  )(pa, pb, gwa0, gwa1, gwb, gb)


def _global_res_body(pa_ref, pb_ref, x_ref, gwa0_ref, gwa1_ref, gwb_ref,
                     gb_ref, scw_ref, scb_ref, o_ref):
  b = pb_ref[0] + pb_ref[1]
  h = (pa_ref[0] @ gwa0_ref[...] + pa_ref[1] @ gwa1_ref[...]
       + b @ gwb_ref[...] + gb_ref[...])
  h = jnp.maximum(h, 0.0)
  sc = x_ref[...] @ scw_ref[...] + scb_ref[...]
  o_ref[...] = jnp.maximum(h + sc, 0.0)


def _tc_global_res(pa, pb, x, gwa0, gwa1, gwb, gb, scw, scb, block=1000):
  _, n, dh = pa.shape
  db = pb.shape[2]
  d = x.shape[1]
  dout = gwa0.shape[1]
  grid = n // block
  full = lambda i: (0, 0)
  return pl.pallas_call(
      _global_res_body,
      grid=(grid,),
      in_specs=[
          pl.BlockSpec((NC, block, dh), lambda i: (0, i, 0)),
          pl.BlockSpec((NC, block, db), lambda i: (0, i, 0)),
          pl.BlockSpec((block, d), lambda i: (i, 0)),
          pl.BlockSpec((dh, dout), full),
          pl.BlockSpec((dh, dout), full),
          pl.BlockSpec((db, dout), full),
          pl.BlockSpec((1, dout), full),
          pl.BlockSpec((d, dout), full),
          pl.BlockSpec((1, dout), full),
      ],
      out_specs=pl.BlockSpec((block, dout), lambda i: (i, 0)),
      out_shape=jax.ShapeDtypeStruct((n, dout), _f32),
  )(pa, pb, x, gwa0, gwa1, gwb, gb, scw, scb)


def _prep_pp(ppW):
  wpi = jnp.zeros((16, ppW.shape[1]), _f32).at[:3].set(ppW[0:3] + ppW[6:9])
  wpj = jnp.zeros((16, ppW.shape[1]), _f32).at[:3].set(ppW[3:6] - ppW[6:9])
  w9 = ppW[9:10]
  return wpi, wpj, w9


def kernel(x, pos, edge_index, ppW1, ppb1, aW1, ab1, gW1, gb1,
           ppW2, ppb2, aW2, ab2, gW2, gb2, scW, scb):
  n, d = x.shape
  src = edge_index[0]
  dst = edge_index[1]
  pos16 = jnp.zeros((n, 16), _f32).at[:, :3].set(pos)

  pj16, pi16 = _sc_gather_pos(pos16, src, dst)
  xj = _sc_gather1(x, src)

  dh = d // 2
  wpi1, wpj1, w91 = _prep_pp(ppW1)
  ma1, mb1 = _tc_edge(xj, pi16, pj16, wpi1, wpj1, w91, ppb1[None, :],
                      aW1[:d], aW1[d:], ab1[None, :])
  pa1, pb1 = _sc_scatter(ma1, mb1, dst, n)
  h1 = _tc_global(pa1, pb1, gW1[:dh], gW1[dh:d], gW1[d:], gb1[None, :])

  hj = _sc_gather1(h1, src)
  wpi2, wpj2, w92 = _prep_pp(ppW2)
  ma2, mb2 = _tc_edge(hj, pi16, pj16, wpi2, wpj2, w92, ppb2[None, :],
                      aW2[:d], aW2[d:], ab2[None, :])
  pa2, pb2 = _sc_scatter(ma2, mb2, dst, n)
  out = _tc_global_res(pa2, pb2, x, gW2[:dh], gW2[dh:d], gW2[d:],
                       gb2[None, :], scW, scb[None, :])
  return out
